# baseline (speedup 1.0000x reference)
import functools

import jax
import jax.numpy as jnp
import numpy as np
from jax import lax
from jax.experimental import pallas as pl
from jax.experimental.pallas import tpu as pltpu
from jax.experimental.pallas import tpu_sc as plsc

_TAU = 30.0
_DT = 1.0
_THRESHOLD = np.float32(0.1)
_DECAY = np.float32(np.exp(np.float32(-_DT / _TAU)))
_ONE_MINUS_DECAY = np.float32(1.0) - _DECAY
_EPS = np.float32(1e-8)

_NUM_CORES = 2
_NUM_SUBCORES = 16
_NW = _NUM_CORES * _NUM_SUBCORES
_K = 1024
_LANES = 16
_UNROLL = 2


_HALO = 4161


@functools.lru_cache(maxsize=None)
def _build_sc_kernel(n_pad_spk, n_nodes, e_pad, n_chunks):
    P = n_chunks * _K
    n_slice = n_nodes // _NUM_SUBCORES
    span = P // 7 + 64
    sw = -(-(span + 16) // 16) * 16
    w_win = -(-(sw + 2 * _HALO + 16) // 128) * 128
    n_mrows = w_win // 128

    mesh = plsc.VectorSubcoreMesh(core_axis_name="c", subcore_axis_name="s")

    @functools.partial(
        pl.kernel,
        mesh=mesh,
        out_type=jax.ShapeDtypeStruct((_NUM_CORES, n_nodes), jnp.float32),
        compiler_params=pltpu.CompilerParams(needs_layout_passes=False),
        scratch_types=[
            pltpu.VMEM((sw,), jnp.float32),
            pltpu.VMEM((2, _K), jnp.int32),
            pltpu.VMEM((2, _K), jnp.int32),
            pltpu.VMEM((2, _K), jnp.int32),
            pltpu.VMEM((w_win,), jnp.float32),
            pltpu.VMEM((n_mrows, 128), jnp.int32),
            pltpu.VMEM((48,), jnp.int32),
            pltpu.VMEM((n_slice,), jnp.float32),
            pltpu.VMEM_SHARED((n_nodes,), jnp.float32),
            pltpu.SemaphoreType.DMA,
            pltpu.SemaphoreType.DMA,
            pltpu.SemaphoreType.DMA,
            pltpu.SemaphoreType.DMA,
        ],
    )
    def sc_edges(spk_hbm, src_hbm, dst_hbm, val_hbm, base_hbm, out_hbm,
                 spk_v, src_v, dst_v, val_v, win_v, idx_v, b_v, zb_v, acc_sh,
                 sem_in0, sem_in1, sem_spk, sem_mg):
        sems_in = (sem_in0, sem_in1)
        cid = lax.axis_index("c")
        sid = lax.axis_index("s")
        wid = cid * _NUM_SUBCORES + sid

        def in_descs(c, b2):
            off = wid * P + c * _K
            sem = sems_in[b2]
            return (
                pltpu.make_async_copy(src_hbm.at[pl.ds(off, _K)],
                                      src_v.at[b2], sem),
                pltpu.make_async_copy(val_hbm.at[pl.ds(off, _K)],
                                      val_v.at[b2], sem),
                pltpu.make_async_copy(dst_hbm.at[pl.ds(off, _K)],
                                      dst_v.at[b2], sem),
            )

        def fire_in(c, b2):
            for d in in_descs(c, b2):
                d.start()

        def wait_in(c, b2):
            for d in in_descs(c, b2):
                d.wait()

        pltpu.sync_copy(base_hbm, b_v)
        s0 = b_v[pl.ds(wid, _LANES)][0]
        sbase = pl.multiple_of(jnp.bitwise_and(s0, jnp.int32(-8)), 8)
        abase = jnp.bitwise_and(
            jnp.maximum(s0 - jnp.int32(_HALO + 7), jnp.int32(0)),
            jnp.int32(-8))
        spk_desc = pltpu.make_async_copy(
            spk_hbm.at[pl.ds(sbase, sw)], spk_v, sem_spk)
        spk_desc.start()
        fire_in(0, 0)

        zeros16 = jnp.zeros((_LANES,), jnp.float32)
        ones16 = jnp.ones((_LANES,), jnp.int32)
        iota16 = lax.iota(jnp.int32, _LANES)
        nmax16 = jnp.full((_LANES,), n_nodes - 1, jnp.int32)
        sbase16 = jnp.full((_LANES,), 1, jnp.int32) * sbase
        abase16 = jnp.full((_LANES,), 1, jnp.int32) * abase

        def zb_body(i, carry):
            zb_v[pl.ds(i * _LANES, _LANES)] = zeros16
            return carry

        lax.fori_loop(0, n_slice // _LANES, zb_body, 0)
        pltpu.sync_copy(zb_v, acc_sh.at[pl.ds(sid * n_slice, n_slice)])

        def wz_body(i, carry):
            o = i * _LANES
            win_v[pl.ds(o, _LANES)] = zeros16
            idx_v[i // 8, pl.ds((i % 8) * _LANES, _LANES)] = jnp.minimum(
                abase16 + o + iota16, nmax16)
            return carry

        lax.fori_loop(0, w_win // _LANES, wz_body, 0)
        spk_desc.wait()

        def chunk_step(c, b2):
            wait_in(c, b2)

            @pl.when(c + 1 < n_chunks)
            def _():
                fire_in(c + 1, 1 - b2)

            for i in range(_K // _LANES):
                sl = pl.ds(i * _LANES, _LANES)
                s = plsc.load_gather(spk_v, [src_v[b2, sl] - sbase16])
                w = lax.shift_left(ones16, val_v[b2, sl]).astype(jnp.float32)
                plsc.addupdate_scatter(win_v, [dst_v[b2, sl] - abase16],
                                       w * (s + _EPS))

        def group_body(g, carry):
            base = g * 2
            chunk_step(base, 0)
            chunk_step(base + 1, 1)
            return carry

        lax.fori_loop(0, n_chunks // 2, group_body, 0)
        plsc.subcore_barrier()

        def mg_desc(j):
            return pltpu.make_async_copy(
                win_v.at[pl.ds(j * 128, 128)],
                acc_sh.at[idx_v.at[j]], sem_mg)

        for j in range(n_mrows):
            mg_desc(j).start(add=True)
            if j >= 32:
                mg_desc(j - 32).wait()
        for j in range(max(0, n_mrows - 32), n_mrows):
            mg_desc(j).wait()
        plsc.subcore_barrier()

        pltpu.sync_copy(acc_sh.at[pl.ds(sid * n_slice, n_slice)],
                        out_hbm.at[cid, pl.ds(sid * n_slice, n_slice)])

    return sc_edges


def _round_f32_to_f16_f32(x):
    t = lax.bitcast_convert_type(x, jnp.int32)
    lsb = jnp.bitwise_and(lax.shift_right_logical(t, 13), jnp.int32(1))
    rn = jnp.bitwise_and(t + lsb + jnp.int32(0x0FFF), jnp.int32(-8192))
    normal = lax.bitcast_convert_type(rn, jnp.float32)
    y = x * jnp.float32(16777216.0)
    sub = ((y + jnp.float32(8388608.0)) - jnp.float32(8388608.0)) * jnp.float32(
        5.9604644775390625e-08)
    return jnp.where(jnp.abs(x) < jnp.float32(6.103515625e-05), sub, normal)


def _lif_body(parts_ref, cur_ref, v_ref, vout_ref, spk_ref):
    total = parts_ref[0] + parts_ref[1]
    syn = _round_f32_to_f16_f32(total)
    current = syn + cur_ref[...]
    v_new = v_ref[...] * _DECAY + current * jnp.float32(_TAU) * _ONE_MINUS_DECAY
    spk = (v_new >= _THRESHOLD).astype(jnp.float32)
    vout_ref[...] = v_new * (jnp.float32(1.0) - spk)
    spk_ref[...] = spk


def kernel(input_current, prev_spikes, v, src_ids, dst_ids, values_exp):
    n = input_current.shape[0]
    e = src_ids.shape[0]
    assert n % (_NUM_SUBCORES * 128) == 0

    n_chunks = -(-e // (_NW * _K))
    n_chunks = -(-n_chunks // _UNROLL) * _UNROLL
    e_pad = _NW * n_chunks * _K
    pad = e_pad - e
    P = n_chunks * _K
    span = P // 7 + 64
    sw = -(-(span + 16) // 16) * 16
    n_pad_spk = n + sw + _LANES

    spk_pad = jnp.concatenate(
        [prev_spikes.astype(jnp.float32),
         jnp.full((sw + _LANES,), -_EPS, jnp.float32)])
    src_p = jnp.concatenate(
        [src_ids.astype(jnp.int32), jnp.full((pad,), n, jnp.int32)])
    dst_p = jnp.concatenate(
        [dst_ids.astype(jnp.int32), jnp.full((pad,), n - 1, jnp.int32)])
    val_p = jnp.concatenate(
        [values_exp.astype(jnp.int32), jnp.zeros((pad,), jnp.int32)])
    bases = jnp.concatenate(
        [src_p.reshape(_NW, P)[:, 0], jnp.zeros((16,), jnp.int32)])

    sc_edges = _build_sc_kernel(n_pad_spk, n, e_pad, n_chunks)
    parts = sc_edges(spk_pad, src_p, dst_p, val_p, bases)

    rows = n // 128
    parts2 = parts.reshape(_NUM_CORES, rows, 128)
    cur2 = input_current.reshape(rows, 128)
    v2 = v.reshape(rows, 128)
    v_out, spikes = pl.pallas_call(
        _lif_body,
        out_shape=(jax.ShapeDtypeStruct((rows, 128), jnp.float32),
                   jax.ShapeDtypeStruct((rows, 128), jnp.float32)),
    )(parts2, cur2, v2)
    return v_out.reshape(n), spikes.reshape(n)


# speedup vs baseline: 1.5526x; 1.5526x over previous
import functools

import jax
import jax.numpy as jnp
import numpy as np
from jax import lax
from jax.experimental import pallas as pl
from jax.experimental.pallas import tpu as pltpu
from jax.experimental.pallas import tpu_sc as plsc

_TAU = 30.0
_DT = 1.0
_THRESHOLD = np.float32(0.1)
_DECAY = np.float32(np.exp(np.float32(-_DT / _TAU)))
_ONE_MINUS_DECAY = np.float32(1.0) - _DECAY
_EPS = np.float32(1e-8)

_NUM_CORES = 2
_NUM_SUBCORES = 16
_NW = _NUM_CORES * _NUM_SUBCORES
_K = 1024
_LANES = 16
_UNROLL = 6


@functools.lru_cache(maxsize=None)
def _build_sc_kernel(n_pad_spk, n_nodes, e_pad, n_chunks):
    P = n_chunks * _K
    n_slice = n_nodes // _NUM_SUBCORES
    n_rows = _K // 128

    mesh = plsc.VectorSubcoreMesh(core_axis_name="c", subcore_axis_name="s")

    @functools.partial(
        pl.kernel,
        mesh=mesh,
        out_type=jax.ShapeDtypeStruct((_NUM_CORES, n_nodes), jnp.float32),
        compiler_params=pltpu.CompilerParams(needs_layout_passes=False),
        scratch_types=[
            pltpu.VMEM((n_pad_spk,), jnp.float32),
            pltpu.VMEM((2, _K), jnp.int32),
            pltpu.VMEM((_K,), jnp.int32),
            pltpu.VMEM((_K,), jnp.int32),
            pltpu.VMEM((_K,), jnp.int32),
            pltpu.VMEM((2, _K), jnp.int32),
            pltpu.VMEM((_K,), jnp.float32),
            pltpu.VMEM((_K,), jnp.float32),
            pltpu.VMEM((_K,), jnp.float32),
            pltpu.VMEM((n_slice,), jnp.float32),
            pltpu.VMEM_SHARED((n_nodes,), jnp.float32),
            pltpu.SemaphoreType.DMA,
            pltpu.SemaphoreType.DMA,
            pltpu.SemaphoreType.DMA,
            pltpu.SemaphoreType.DMA,
            pltpu.SemaphoreType.DMA,
        ],
    )
    def sc_edges(spk_hbm, src_hbm, dst_hbm, val_hbm, out_hbm,
                 spk_v, src_v, dst_v0, dst_v1, dst_v2, val_v,
                 contrib_v0, contrib_v1, contrib_v2, zb_v, acc_sh,
                 sem_in0, sem_in1, sem_sc0, sem_sc1, sem_sc2):
        sems_in = (sem_in0, sem_in1)
        sems_sc = (sem_sc0, sem_sc1, sem_sc2)
        dst_vs = (dst_v0, dst_v1, dst_v2)
        contrib_vs = (contrib_v0, contrib_v1, contrib_v2)
        cid = lax.axis_index("c")
        sid = lax.axis_index("s")
        wid = cid * _NUM_SUBCORES + sid

        pltpu.sync_copy(spk_hbm, spk_v)

        zeros16 = jnp.zeros((_LANES,), jnp.float32)

        def zb_body(i, carry):
            zb_v[pl.ds(i * _LANES, _LANES)] = zeros16
            return carry

        lax.fori_loop(0, n_slice // _LANES, zb_body, 0)
        pltpu.sync_copy(zb_v, acc_sh.at[pl.ds(sid * n_slice, n_slice)])
        plsc.subcore_barrier()

        ones16 = jnp.ones((_LANES,), jnp.int32)

        def in_descs(c, b2, b3):
            off = wid * P + c * _K
            sem = sems_in[b2]
            return (
                pltpu.make_async_copy(src_hbm.at[pl.ds(off, _K)],
                                      src_v.at[b2], sem),
                pltpu.make_async_copy(val_hbm.at[pl.ds(off, _K)],
                                      val_v.at[b2], sem),
                pltpu.make_async_copy(dst_hbm.at[pl.ds(off, _K)],
                                      dst_vs[b3], sem),
            )

        def sc_descs(b3):
            sem = sems_sc[b3]
            return (
                pltpu.make_async_copy(contrib_vs[b3],
                                      acc_sh.at[dst_vs[b3]], sem),)

        def fire_in(c, b2, b3):
            for d in in_descs(c, b2, b3):
                d.start()

        def wait_in(c, b2, b3):
            for d in in_descs(c, b2, b3):
                d.wait()

        def fire_scatter(b3):
            for d in sc_descs(b3):
                d.start(add=True)

        def drain_scatter(b3):
            for d in sc_descs(b3):
                d.wait()

        fire_in(0, 0, 0)

        def chunk_step(c, b2, b3):
            wait_in(c, b2, b3)

            @pl.when(c >= 2)
            def _():
                drain_scatter((b3 + 1) % 3)

            @pl.when(c + 1 < n_chunks)
            def _():
                fire_in(c + 1, 1 - b2, (b3 + 1) % 3)

            for i in range(_K // _LANES):
                sl = pl.ds(i * _LANES, _LANES)
                s = plsc.load_gather(spk_v, [src_v[b2, sl]])
                w = lax.shift_left(ones16, val_v[b2, sl]).astype(jnp.float32)
                contrib_vs[b3][sl] = w * (s + _EPS)
            fire_scatter(b3)

        def group_body(g, carry):
            base = g * _UNROLL
            for u in range(_UNROLL):
                chunk_step(base + u, u % 2, u % 3)
            return carry

        lax.fori_loop(0, n_chunks // _UNROLL, group_body, 0)
        drain_scatter((n_chunks - 2) % 3)
        drain_scatter((n_chunks - 1) % 3)
        plsc.subcore_barrier()

        pltpu.sync_copy(acc_sh.at[pl.ds(sid * n_slice, n_slice)],
                        out_hbm.at[cid, pl.ds(sid * n_slice, n_slice)])

    return sc_edges


def _round_f32_to_f16_f32(x):
    t = lax.bitcast_convert_type(x, jnp.int32)
    lsb = jnp.bitwise_and(lax.shift_right_logical(t, 13), jnp.int32(1))
    rn = jnp.bitwise_and(t + lsb + jnp.int32(0x0FFF), jnp.int32(-8192))
    normal = lax.bitcast_convert_type(rn, jnp.float32)
    y = x * jnp.float32(16777216.0)
    sub = ((y + jnp.float32(8388608.0)) - jnp.float32(8388608.0)) * jnp.float32(
        5.9604644775390625e-08)
    return jnp.where(jnp.abs(x) < jnp.float32(6.103515625e-05), sub, normal)


def _lif_body(parts_ref, cur_ref, v_ref, vout_ref, spk_ref):
    total = parts_ref[0] + parts_ref[1]
    syn = _round_f32_to_f16_f32(total)
    current = syn + cur_ref[...]
    v_new = v_ref[...] * _DECAY + current * jnp.float32(_TAU) * _ONE_MINUS_DECAY
    spk = (v_new >= _THRESHOLD).astype(jnp.float32)
    vout_ref[...] = v_new * (jnp.float32(1.0) - spk)
    spk_ref[...] = spk


def kernel(input_current, prev_spikes, v, src_ids, dst_ids, values_exp):
    n = input_current.shape[0]
    e = src_ids.shape[0]
    assert n % (_NUM_SUBCORES * 128) == 0

    n_chunks = -(-e // (_NW * _K))
    n_chunks = -(-n_chunks // _UNROLL) * _UNROLL
    e_pad = _NW * n_chunks * _K
    pad = e_pad - e
    n_pad_spk = n + _LANES

    spk_pad = jnp.concatenate(
        [prev_spikes.astype(jnp.float32),
         jnp.full((_LANES,), -_EPS, jnp.float32)])
    src_p = jnp.concatenate(
        [src_ids.astype(jnp.int32), jnp.full((pad,), n, jnp.int32)])
    dst_p = jnp.concatenate(
        [dst_ids.astype(jnp.int32), jnp.zeros((pad,), jnp.int32)])
    val_p = jnp.concatenate(
        [values_exp.astype(jnp.int32), jnp.zeros((pad,), jnp.int32)])

    sc_edges = _build_sc_kernel(n_pad_spk, n, e_pad, n_chunks)
    parts = sc_edges(spk_pad, src_p, dst_p, val_p)

    rows = n // 128
    parts2 = parts.reshape(_NUM_CORES, rows, 128)
    cur2 = input_current.reshape(rows, 128)
    v2 = v.reshape(rows, 128)
    v_out, spikes = pl.pallas_call(
        _lif_body,
        out_shape=(jax.ShapeDtypeStruct((rows, 128), jnp.float32),
                   jax.ShapeDtypeStruct((rows, 128), jnp.float32)),
    )(parts2, cur2, v2)
    return v_out.reshape(n), spikes.reshape(n)


# speedup vs baseline: 1.6740x; 1.0782x over previous
import functools

import jax
import jax.numpy as jnp
import numpy as np
from jax import lax
from jax.experimental import pallas as pl
from jax.experimental.pallas import tpu as pltpu
from jax.experimental.pallas import tpu_sc as plsc

_TAU = 30.0
_DT = 1.0
_THRESHOLD = np.float32(0.1)
_DECAY = np.float32(np.exp(np.float32(-_DT / _TAU)))
_ONE_MINUS_DECAY = np.float32(1.0) - _DECAY
_EPS = np.float32(1e-8)

_NUM_CORES = 2
_NUM_SUBCORES = 16
_NW = _NUM_CORES * _NUM_SUBCORES
_K = 1024
_LANES = 16
_UNROLL = 6


@functools.lru_cache(maxsize=None)
def _build_sc_kernel(n_pad_spk, n_nodes, e, n_chunks):
    P = n_chunks * _K
    n_slice = n_nodes // _NUM_SUBCORES
    n_tail = -(-(e - (_NW - 1) * P) // _K)

    mesh = plsc.VectorSubcoreMesh(core_axis_name="c", subcore_axis_name="s")

    @functools.partial(
        pl.kernel,
        mesh=mesh,
        out_type=jax.ShapeDtypeStruct((_NUM_CORES, n_nodes), jnp.float32),
        compiler_params=pltpu.CompilerParams(needs_layout_passes=False),
        scratch_types=[
            pltpu.VMEM((n_pad_spk,), jnp.float32),
            pltpu.VMEM((2, _K), jnp.int32),
            pltpu.VMEM((_K,), jnp.int32),
            pltpu.VMEM((_K,), jnp.int32),
            pltpu.VMEM((_K,), jnp.int32),
            pltpu.VMEM((2, _K), jnp.int32),
            pltpu.VMEM((_K,), jnp.float32),
            pltpu.VMEM((_K,), jnp.float32),
            pltpu.VMEM((_K,), jnp.float32),
            pltpu.VMEM((n_slice,), jnp.float32),
            pltpu.VMEM_SHARED((n_nodes,), jnp.float32),
            pltpu.SemaphoreType.DMA,
            pltpu.SemaphoreType.DMA,
            pltpu.SemaphoreType.DMA,
            pltpu.SemaphoreType.DMA,
            pltpu.SemaphoreType.DMA,
        ],
    )
    def sc_edges(spk_hbm, src_hbm, dst_hbm, val_hbm,
                 tsrc_hbm, tdst_hbm, tval_hbm, out_hbm,
                 spk_v, src_v, dst_v0, dst_v1, dst_v2, val_v,
                 contrib_v0, contrib_v1, contrib_v2, zb_v, acc_sh,
                 sem_in0, sem_in1, sem_sc0, sem_sc1, sem_sc2):
        sems_in = (sem_in0, sem_in1)
        sems_sc = (sem_sc0, sem_sc1, sem_sc2)
        dst_vs = (dst_v0, dst_v1, dst_v2)
        contrib_vs = (contrib_v0, contrib_v1, contrib_v2)
        cid = lax.axis_index("c")
        sid = lax.axis_index("s")
        wid = cid * _NUM_SUBCORES + sid
        is_last = wid == (_NW - 1)

        pltpu.sync_copy(spk_hbm, spk_v)

        zeros16 = jnp.zeros((_LANES,), jnp.float32)

        def zb_body(i, carry):
            zb_v[pl.ds(i * _LANES, _LANES)] = zeros16
            return carry

        lax.fori_loop(0, n_slice // _LANES, zb_body, 0)
        pltpu.sync_copy(zb_v, acc_sh.at[pl.ds(sid * n_slice, n_slice)])
        plsc.subcore_barrier()

        ones16 = jnp.ones((_LANES,), jnp.int32)

        def in_descs(c, b2, b3):
            off = wid * P + c * _K
            sem = sems_in[b2]
            return (
                pltpu.make_async_copy(src_hbm.at[pl.ds(off, _K)],
                                      src_v.at[b2], sem),
                pltpu.make_async_copy(val_hbm.at[pl.ds(off, _K)],
                                      val_v.at[b2], sem),
                pltpu.make_async_copy(dst_hbm.at[pl.ds(off, _K)],
                                      dst_vs[b3], sem),
            )

        def in_descs_tail(c, b2, b3):
            off = c * _K
            sem = sems_in[b2]
            return (
                pltpu.make_async_copy(tsrc_hbm.at[pl.ds(off, _K)],
                                      src_v.at[b2], sem),
                pltpu.make_async_copy(tval_hbm.at[pl.ds(off, _K)],
                                      val_v.at[b2], sem),
                pltpu.make_async_copy(tdst_hbm.at[pl.ds(off, _K)],
                                      dst_vs[b3], sem),
            )

        def chunk_valid(c):
            return jnp.logical_or(jnp.logical_not(is_last), c < n_tail)

        def sc_descs(b3):
            sem = sems_sc[b3]
            return (
                pltpu.make_async_copy(contrib_vs[b3],
                                      acc_sh.at[dst_vs[b3]], sem),)

        def fire_in(c, b2, b3):
            @pl.when(jnp.logical_not(is_last))
            def _():
                for d in in_descs(c, b2, b3):
                    d.start()

            @pl.when(jnp.logical_and(is_last, c < n_tail))
            def _():
                for d in in_descs_tail(c, b2, b3):
                    d.start()

        def wait_in(c, b2, b3):
            @pl.when(jnp.logical_not(is_last))
            def _():
                for d in in_descs(c, b2, b3):
                    d.wait()

            @pl.when(jnp.logical_and(is_last, c < n_tail))
            def _():
                for d in in_descs_tail(c, b2, b3):
                    d.wait()

        def fire_scatter(b3):
            for d in sc_descs(b3):
                d.start(add=True)

        def drain_scatter(b3):
            for d in sc_descs(b3):
                d.wait()

        fire_in(0, 0, 0)

        def chunk_step(c, b2, b3):
            wait_in(c, b2, b3)

            @pl.when(jnp.logical_and(c >= 2, chunk_valid(c - 2)))
            def _():
                drain_scatter((b3 + 1) % 3)

            @pl.when(c + 1 < n_chunks)
            def _():
                fire_in(c + 1, 1 - b2, (b3 + 1) % 3)

            @pl.when(chunk_valid(c))
            def _():
                for i in range(_K // _LANES):
                    sl = pl.ds(i * _LANES, _LANES)
                    s = plsc.load_gather(spk_v, [src_v[b2, sl]])
                    w = lax.shift_left(ones16,
                                       val_v[b2, sl]).astype(jnp.float32)
                    contrib_vs[b3][sl] = w * (s + _EPS)
                fire_scatter(b3)

        def group_body(g, carry):
            base = g * _UNROLL
            for u in range(_UNROLL):
                chunk_step(base + u, u % 2, u % 3)
            return carry

        lax.fori_loop(0, n_chunks // _UNROLL, group_body, 0)

        @pl.when(jnp.logical_not(is_last))
        def _():
            drain_scatter((n_chunks - 2) % 3)
            drain_scatter((n_chunks - 1) % 3)

        plsc.subcore_barrier()

        pltpu.sync_copy(acc_sh.at[pl.ds(sid * n_slice, n_slice)],
                        out_hbm.at[cid, pl.ds(sid * n_slice, n_slice)])

    return sc_edges


def _round_f32_to_f16_f32(x):
    t = lax.bitcast_convert_type(x, jnp.int32)
    lsb = jnp.bitwise_and(lax.shift_right_logical(t, 13), jnp.int32(1))
    rn = jnp.bitwise_and(t + lsb + jnp.int32(0x0FFF), jnp.int32(-8192))
    normal = lax.bitcast_convert_type(rn, jnp.float32)
    y = x * jnp.float32(16777216.0)
    sub = ((y + jnp.float32(8388608.0)) - jnp.float32(8388608.0)) * jnp.float32(
        5.9604644775390625e-08)
    return jnp.where(jnp.abs(x) < jnp.float32(6.103515625e-05), sub, normal)


def _lif_body(parts_ref, cur_ref, v_ref, vout_ref, spk_ref):
    total = parts_ref[0] + parts_ref[1]
    syn = _round_f32_to_f16_f32(total)
    current = syn + cur_ref[...]
    v_new = v_ref[...] * _DECAY + current * jnp.float32(_TAU) * _ONE_MINUS_DECAY
    spk = (v_new >= _THRESHOLD).astype(jnp.float32)
    vout_ref[...] = v_new * (jnp.float32(1.0) - spk)
    spk_ref[...] = spk


def kernel(input_current, prev_spikes, v, src_ids, dst_ids, values_exp):
    n = input_current.shape[0]
    e = src_ids.shape[0]
    assert n % (_NUM_SUBCORES * 128) == 0

    n_chunks = -(-e // (_NW * _K))
    n_chunks = -(-n_chunks // _UNROLL) * _UNROLL
    n_pad_spk = n + _LANES
    P = n_chunks * _K
    tail_lo = (_NW - 1) * P
    n_tail = -(-(e - tail_lo) // _K)
    tpad = tail_lo + n_tail * _K - e

    spk_pad = jnp.concatenate(
        [prev_spikes.astype(jnp.float32),
         jnp.full((_LANES,), -_EPS, jnp.float32)])
    src_i = src_ids.astype(jnp.int32)
    dst_i = dst_ids.astype(jnp.int32)
    val_i = values_exp.astype(jnp.int32)
    tsrc = jnp.concatenate([src_i[tail_lo:], jnp.full((tpad,), n, jnp.int32)])
    tdst = jnp.concatenate(
        [dst_i[tail_lo:], jnp.full((tpad,), n - 1, jnp.int32)])
    tval = jnp.concatenate([val_i[tail_lo:], jnp.zeros((tpad,), jnp.int32)])

    sc_edges = _build_sc_kernel(n_pad_spk, n, e, n_chunks)
    parts = sc_edges(spk_pad, src_i, dst_i, val_i, tsrc, tdst, tval)

    rows = n // 128
    parts2 = parts.reshape(_NUM_CORES, rows, 128)
    cur2 = input_current.reshape(rows, 128)
    v2 = v.reshape(rows, 128)
    v_out, spikes = pl.pallas_call(
        _lif_body,
        out_shape=(jax.ShapeDtypeStruct((rows, 128), jnp.float32),
                   jax.ShapeDtypeStruct((rows, 128), jnp.float32)),
    )(parts2, cur2, v2)
    return v_out.reshape(n), spikes.reshape(n)


# speedup vs baseline: 1.7207x; 1.0279x over previous
import functools

import jax
import jax.numpy as jnp
import numpy as np
from jax import lax
from jax.experimental import pallas as pl
from jax.experimental.pallas import tpu as pltpu
from jax.experimental.pallas import tpu_sc as plsc

_TAU = 30.0
_DT = 1.0
_THRESHOLD = np.float32(0.1)
_DECAY = np.float32(np.exp(np.float32(-_DT / _TAU)))
_ONE_MINUS_DECAY = np.float32(1.0) - _DECAY
_EPS = np.float32(1e-8)

_NUM_CORES = 2
_NUM_SUBCORES = 16
_NW = _NUM_CORES * _NUM_SUBCORES
_K = 1024
_LANES = 16
_UNROLL = 6


@functools.lru_cache(maxsize=None)
def _build_sc_kernel(n_pad_spk, n_nodes, e, n_chunks):
    P = n_chunks * _K
    n_slice = n_nodes // _NUM_SUBCORES
    n_tail = -(-(e - (_NW - 1) * P) // _K)

    mesh = plsc.VectorSubcoreMesh(core_axis_name="c", subcore_axis_name="s")

    @functools.partial(
        pl.kernel,
        mesh=mesh,
        out_type=jax.ShapeDtypeStruct((_NUM_CORES, n_nodes), jnp.float32),
        compiler_params=pltpu.CompilerParams(needs_layout_passes=False),
        scratch_types=[
            pltpu.VMEM((n_pad_spk,), jnp.float32),
            pltpu.VMEM((2, _K), jnp.int32),
            pltpu.VMEM((_K,), jnp.int32),
            pltpu.VMEM((_K,), jnp.int32),
            pltpu.VMEM((_K,), jnp.int32),
            pltpu.VMEM((2, _K), jnp.int32),
            pltpu.VMEM((_K,), jnp.float32),
            pltpu.VMEM((_K,), jnp.float32),
            pltpu.VMEM((_K,), jnp.float32),
            pltpu.VMEM((n_slice,), jnp.float32),
            pltpu.VMEM_SHARED((n_nodes,), jnp.float32),
            pltpu.SemaphoreType.DMA,
            pltpu.SemaphoreType.DMA,
            pltpu.SemaphoreType.DMA,
            pltpu.SemaphoreType.DMA,
            pltpu.SemaphoreType.DMA,
            pltpu.SemaphoreType.DMA,
        ],
    )
    def sc_edges(spk_hbm, src_hbm, dst_hbm, val_hbm,
                 tsrc_hbm, tdst_hbm, tval_hbm, out_hbm,
                 spk_v, src_v, dst_v0, dst_v1, dst_v2, val_v,
                 contrib_v0, contrib_v1, contrib_v2, zb_v, acc_sh,
                 sem_in0, sem_in1, sem_sc0, sem_sc1, sem_sc2, sem_spk):
        sems_in = (sem_in0, sem_in1)
        sems_sc = (sem_sc0, sem_sc1, sem_sc2)
        dst_vs = (dst_v0, dst_v1, dst_v2)
        contrib_vs = (contrib_v0, contrib_v1, contrib_v2)
        cid = lax.axis_index("c")
        sid = lax.axis_index("s")
        wid = cid * _NUM_SUBCORES + sid
        is_last = wid == (_NW - 1)

        spk_desc = pltpu.make_async_copy(spk_hbm, spk_v, sem_spk)
        spk_desc.start()

        zeros16 = jnp.zeros((_LANES,), jnp.float32)

        def zb_body(i, carry):
            zb_v[pl.ds(i * _LANES, _LANES)] = zeros16
            return carry

        lax.fori_loop(0, n_slice // _LANES, zb_body, 0)
        pltpu.sync_copy(zb_v, acc_sh.at[pl.ds(sid * n_slice, n_slice)])
        plsc.subcore_barrier()

        ones16 = jnp.ones((_LANES,), jnp.int32)

        def in_descs(c, b2, b3):
            off = wid * P + c * _K
            sem = sems_in[b2]
            return (
                pltpu.make_async_copy(src_hbm.at[pl.ds(off, _K)],
                                      src_v.at[b2], sem),
                pltpu.make_async_copy(val_hbm.at[pl.ds(off, _K)],
                                      val_v.at[b2], sem),
                pltpu.make_async_copy(dst_hbm.at[pl.ds(off, _K)],
                                      dst_vs[b3], sem),
            )

        def in_descs_tail(c, b2, b3):
            off = c * _K
            sem = sems_in[b2]
            return (
                pltpu.make_async_copy(tsrc_hbm.at[pl.ds(off, _K)],
                                      src_v.at[b2], sem),
                pltpu.make_async_copy(tval_hbm.at[pl.ds(off, _K)],
                                      val_v.at[b2], sem),
                pltpu.make_async_copy(tdst_hbm.at[pl.ds(off, _K)],
                                      dst_vs[b3], sem),
            )

        def chunk_valid(c):
            return jnp.logical_or(jnp.logical_not(is_last), c < n_tail)

        def sc_descs(b3):
            sem = sems_sc[b3]
            return (
                pltpu.make_async_copy(contrib_vs[b3],
                                      acc_sh.at[dst_vs[b3]], sem),)

        def fire_in(c, b2, b3):
            @pl.when(jnp.logical_not(is_last))
            def _():
                for d in in_descs(c, b2, b3):
                    d.start()

            @pl.when(jnp.logical_and(is_last, c < n_tail))
            def _():
                for d in in_descs_tail(c, b2, b3):
                    d.start()

        def wait_in(c, b2, b3):
            @pl.when(jnp.logical_not(is_last))
            def _():
                for d in in_descs(c, b2, b3):
                    d.wait()

            @pl.when(jnp.logical_and(is_last, c < n_tail))
            def _():
                for d in in_descs_tail(c, b2, b3):
                    d.wait()

        def fire_scatter(b3):
            for d in sc_descs(b3):
                d.start(add=True)

        def drain_scatter(b3):
            for d in sc_descs(b3):
                d.wait()

        fire_in(0, 0, 0)
        spk_desc.wait()

        def chunk_step(c, b2, b3):
            wait_in(c, b2, b3)

            @pl.when(jnp.logical_and(c >= 2, chunk_valid(c - 2)))
            def _():
                drain_scatter((b3 + 1) % 3)

            @pl.when(c + 1 < n_chunks)
            def _():
                fire_in(c + 1, 1 - b2, (b3 + 1) % 3)

            @pl.when(chunk_valid(c))
            def _():
                for i in range(_K // _LANES):
                    sl = pl.ds(i * _LANES, _LANES)
                    s = plsc.load_gather(spk_v, [src_v[b2, sl]])
                    w = lax.shift_left(ones16,
                                       val_v[b2, sl]).astype(jnp.float32)
                    contrib_vs[b3][sl] = w * (s + _EPS)
                fire_scatter(b3)

        def group_body(g, carry):
            base = g * _UNROLL
            for u in range(_UNROLL):
                chunk_step(base + u, u % 2, u % 3)
            return carry

        lax.fori_loop(0, n_chunks // _UNROLL, group_body, 0)

        @pl.when(jnp.logical_not(is_last))
        def _():
            drain_scatter((n_chunks - 2) % 3)
            drain_scatter((n_chunks - 1) % 3)

        plsc.subcore_barrier()

        pltpu.sync_copy(acc_sh.at[pl.ds(sid * n_slice, n_slice)],
                        out_hbm.at[cid, pl.ds(sid * n_slice, n_slice)])

    return sc_edges


def _round_f32_to_f16_f32(x):
    t = lax.bitcast_convert_type(x, jnp.int32)
    lsb = jnp.bitwise_and(lax.shift_right_logical(t, 13), jnp.int32(1))
    rn = jnp.bitwise_and(t + lsb + jnp.int32(0x0FFF), jnp.int32(-8192))
    normal = lax.bitcast_convert_type(rn, jnp.float32)
    y = x * jnp.float32(16777216.0)
    sub = ((y + jnp.float32(8388608.0)) - jnp.float32(8388608.0)) * jnp.float32(
        5.9604644775390625e-08)
    return jnp.where(jnp.abs(x) < jnp.float32(6.103515625e-05), sub, normal)


def _lif_body(parts_ref, cur_ref, v_ref, vout_ref, spk_ref):
    total = parts_ref[0] + parts_ref[1]
    syn = _round_f32_to_f16_f32(total)
    current = syn + cur_ref[...]
    v_new = v_ref[...] * _DECAY + current * jnp.float32(_TAU) * _ONE_MINUS_DECAY
    spk = (v_new >= _THRESHOLD).astype(jnp.float32)
    vout_ref[...] = v_new * (jnp.float32(1.0) - spk)
    spk_ref[...] = spk


def kernel(input_current, prev_spikes, v, src_ids, dst_ids, values_exp):
    n = input_current.shape[0]
    e = src_ids.shape[0]
    assert n % (_NUM_SUBCORES * 128) == 0

    n_chunks = -(-e // (_NW * _K))
    n_chunks = -(-n_chunks // _UNROLL) * _UNROLL
    n_pad_spk = n + _LANES
    P = n_chunks * _K
    tail_lo = (_NW - 1) * P
    n_tail = -(-(e - tail_lo) // _K)
    tpad = tail_lo + n_tail * _K - e

    spk_pad = jnp.concatenate(
        [prev_spikes.astype(jnp.float32),
         jnp.full((_LANES,), -_EPS, jnp.float32)])
    src_i = src_ids.astype(jnp.int32)
    dst_i = dst_ids.astype(jnp.int32)
    val_i = values_exp.astype(jnp.int32)
    tsrc = jnp.concatenate([src_i[tail_lo:], jnp.full((tpad,), n, jnp.int32)])
    tdst = jnp.concatenate(
        [dst_i[tail_lo:], jnp.full((tpad,), n - 1, jnp.int32)])
    tval = jnp.concatenate([val_i[tail_lo:], jnp.zeros((tpad,), jnp.int32)])

    sc_edges = _build_sc_kernel(n_pad_spk, n, e, n_chunks)
    parts = sc_edges(spk_pad, src_i, dst_i, val_i, tsrc, tdst, tval)

    rows = n // 128
    parts2 = parts.reshape(_NUM_CORES, rows, 128)
    cur2 = input_current.reshape(rows, 128)
    v2 = v.reshape(rows, 128)
    v_out, spikes = pl.pallas_call(
        _lif_body,
        out_shape=(jax.ShapeDtypeStruct((rows, 128), jnp.float32),
                   jax.ShapeDtypeStruct((rows, 128), jnp.float32)),
    )(parts2, cur2, v2)
    return v_out.reshape(n), spikes.reshape(n)
